# baseline (speedup 1.0000x reference)
import jax
import jax.numpy as jnp
import numpy as np
from jax.experimental import pallas as pl
from jax.experimental.pallas import tpu as pltpu

_PIXELS = 4096
_HIDDEN_P = 1280
_LATENT_P = 128
_LATENT = 4

_U32 = jnp.uint32

_ERFINV_CENTRAL = [2.81022636e-08, 3.43273939e-07, -3.5233877e-06,
                   -4.39150654e-06, 0.00021858087, -0.00125372503,
                   -0.00417768164, 0.246640727, 1.50140941]
_ERFINV_TAIL = [-0.000200214257, 0.000100950558, 0.00134934322,
                -0.00367342844, 0.00573950773, -0.0076224613,
                0.00943887047, 1.00167406, 2.83297682]


def _rotl(x, d):
    return (x << _U32(d)) | (x >> _U32(32 - d))


def _threefry2x32(k1, k2, x0, x1):
    ks2 = k1 ^ k2 ^ _U32(0x1BD11BDA)
    x0 = x0 + k1
    x1 = x1 + k2
    r_even = (13, 15, 26, 6)
    r_odd = (17, 29, 16, 24)
    sched = ((r_even, k2, ks2, 1), (r_odd, ks2, k1, 2), (r_even, k1, k2, 3),
             (r_odd, k2, ks2, 4), (r_even, ks2, k1, 5))
    for rots, a0, a1, inc in sched:
        for r in rots:
            x0 = x0 + x1
            x1 = _rotl(x1, r)
            x1 = x0 ^ x1
        x0 = x0 + a0
        x1 = x1 + a1 + _U32(inc)
    return x0, x1


def _eps_tile(key_ref, row0, bm):
    k1 = key_ref[0]
    k2 = key_ref[1]
    r = jax.lax.broadcasted_iota(_U32, (bm, _LATENT_P), 0)
    c = jax.lax.broadcasted_iota(_U32, (bm, _LATENT_P), 1)
    n = _U32(_LATENT) * (row0.astype(_U32) + r) + c
    b0, b1 = _threefry2x32(k1, k2, jnp.zeros_like(n), n)
    bits = b0 ^ b1
    fb = (bits >> _U32(9)) | _U32(0x3F800000)
    f = jax.lax.bitcast_convert_type(fb, jnp.float32) - jnp.float32(1.0)
    lo = np.nextafter(np.float32(-1), np.float32(0), dtype=np.float32)
    u = jnp.maximum(lo, f * (np.float32(1.0) - lo) + lo)
    w = -jnp.log1p(-u * u)
    wa = w - jnp.float32(2.5)
    wb = jnp.sqrt(w) - jnp.float32(3.0)
    pa = jnp.float32(_ERFINV_CENTRAL[0])
    for coef in _ERFINV_CENTRAL[1:]:
        pa = pa * wa + jnp.float32(coef)
    pb = jnp.float32(_ERFINV_TAIL[0])
    for coef in _ERFINV_TAIL[1:]:
        pb = pb * wb + jnp.float32(coef)
    p = jnp.where(w < jnp.float32(5.0), pa, pb)
    eps = np.float32(np.sqrt(2)) * (p * u)
    return jnp.where(c < _U32(_LATENT), eps, jnp.float32(0.0))


def _vae_kernel(key_ref, x_ref, w1_hbm, b1_ref, w2_hbm, b2_ref, w31_ref,
                b31_ref, w32_ref, b32_ref, w4_ref, b4_ref, w5_hbm, b5_ref,
                w6_hbm, b6_ref, recon_ref, mu_ref, lv_ref, z_ref,
                w1_vmem, w2_vmem, w5_vmem, w6_vmem,
                sem1a, sem1b, sem2, sem5, sem6):
    f32 = jnp.float32
    i = pl.program_id(0)
    bm = recon_ref.shape[0]
    kh = _PIXELS // 2

    cp1a = pltpu.make_async_copy(w1_hbm.at[pl.ds(0, kh), :],
                                 w1_vmem.at[pl.ds(0, kh), :], sem1a)
    cp1b = pltpu.make_async_copy(w1_hbm.at[pl.ds(kh, kh), :],
                                 w1_vmem.at[pl.ds(kh, kh), :], sem1b)
    cp2 = pltpu.make_async_copy(w2_hbm, w2_vmem, sem2)
    cp5 = pltpu.make_async_copy(w5_hbm, w5_vmem, sem5)
    cp6 = pltpu.make_async_copy(w6_hbm, w6_vmem, sem6)

    @pl.when(i == 0)
    def _start_weight_fetch():
        cp1a.start()
        cp1b.start()
        cp2.start()
        cp5.start()
        cp6.start()

    xb = x_ref[...].astype(jnp.bfloat16)
    dn = (((0,), (0,)), ((), ()))

    @pl.when(i == 0)
    def _wait_w1a():
        cp1a.wait()

    h = jax.lax.dot_general(xb[:kh], w1_vmem[:kh], dn,
                            preferred_element_type=f32)

    @pl.when(i == 0)
    def _wait_w1b():
        cp1b.wait()

    h = h + jax.lax.dot_general(xb[kh:], w1_vmem[kh:], dn,
                                preferred_element_type=f32)
    h = h + b1_ref[...]
    h = jnp.maximum(h, 0.0).astype(jnp.bfloat16)

    @pl.when(i == 0)
    def _wait_w2_fetch():
        cp2.wait()

    h = jnp.dot(h, w2_vmem[...], preferred_element_type=f32) + b2_ref[...]
    h = jnp.maximum(h, 0.0).astype(jnp.bfloat16)
    mu = jnp.dot(h, w31_ref[...], preferred_element_type=f32) + b31_ref[...]
    lv = jnp.dot(h, w32_ref[...], preferred_element_type=f32) + b32_ref[...]
    eps = _eps_tile(key_ref, i * bm, bm)
    z = mu + eps * jnp.exp(0.5 * lv)
    h4 = jnp.dot(z.astype(jnp.bfloat16), w4_ref[...],
                 preferred_element_type=f32) + b4_ref[...]
    h4 = jnp.maximum(h4, 0.0).astype(jnp.bfloat16)

    @pl.when(i == 0)
    def _wait_decoder_fetch():
        cp5.wait()
        cp6.wait()

    h5 = jnp.dot(h4, w5_vmem[...], preferred_element_type=f32) + b5_ref[...]
    h5 = jnp.maximum(h5, 0.0).astype(jnp.bfloat16)
    recon = jnp.dot(h5, w6_vmem[...], preferred_element_type=f32) + b6_ref[...]
    recon_ref[...] = jax.nn.sigmoid(recon)
    mu_ref[...] = mu[:, :_LATENT]
    lv_ref[...] = lv[:, :_LATENT]
    z_ref[...] = z[:, :_LATENT]


def kernel(fc1_w, fc1_b, fc2_w, fc2_b, fc31_w, fc31_b, fc32_w, fc32_b,
           fc4_w, fc4_b, fc5_w, fc5_b, fc6_w, fc6_b, x, eps_key):
    B = x.shape[0]
    xt = x.reshape(B, _PIXELS).T

    bm = 256
    while B % bm:
        bm //= 2
    grid = (B // bm,)

    row = lambda i: (i, 0)
    col = lambda i: (0, i)
    const = lambda i: (0, 0)

    recon, mu, lv, z = pl.pallas_call(
        _vae_kernel,
        out_shape=(
            jax.ShapeDtypeStruct((B, _PIXELS), jnp.float32),
            jax.ShapeDtypeStruct((B, _LATENT), jnp.float32),
            jax.ShapeDtypeStruct((B, _LATENT), jnp.float32),
            jax.ShapeDtypeStruct((B, _LATENT), jnp.float32),
        ),
        grid=grid,
        in_specs=[
            pl.BlockSpec(memory_space=pltpu.SMEM),
            pl.BlockSpec((_PIXELS, bm), col),
            pl.BlockSpec(memory_space=pl.ANY),
            pl.BlockSpec((1, _HIDDEN_P), const),
            pl.BlockSpec(memory_space=pl.ANY),
            pl.BlockSpec((1, _HIDDEN_P), const),
            pl.BlockSpec((_HIDDEN_P, _LATENT_P), const),
            pl.BlockSpec((1, _LATENT_P), const),
            pl.BlockSpec((_HIDDEN_P, _LATENT_P), const),
            pl.BlockSpec((1, _LATENT_P), const),
            pl.BlockSpec((_LATENT_P, _HIDDEN_P), const),
            pl.BlockSpec((1, _HIDDEN_P), const),
            pl.BlockSpec(memory_space=pl.ANY),
            pl.BlockSpec((1, _HIDDEN_P), const),
            pl.BlockSpec(memory_space=pl.ANY),
            pl.BlockSpec((1, _PIXELS), const),
        ],
        out_specs=(
            pl.BlockSpec((bm, _PIXELS), row),
            pl.BlockSpec((bm, _LATENT), row),
            pl.BlockSpec((bm, _LATENT), row),
            pl.BlockSpec((bm, _LATENT), row),
        ),
        scratch_shapes=[
            pltpu.VMEM((_PIXELS, _HIDDEN_P), jnp.bfloat16),
            pltpu.VMEM((_HIDDEN_P, _HIDDEN_P), jnp.bfloat16),
            pltpu.VMEM((_HIDDEN_P, _HIDDEN_P), jnp.bfloat16),
            pltpu.VMEM((_HIDDEN_P, _PIXELS), jnp.bfloat16),
            pltpu.SemaphoreType.DMA,
            pltpu.SemaphoreType.DMA,
            pltpu.SemaphoreType.DMA,
            pltpu.SemaphoreType.DMA,
            pltpu.SemaphoreType.DMA,
        ],
        compiler_params=pltpu.CompilerParams(
            dimension_semantics=("arbitrary",),
            vmem_limit_bytes=56 * 1024 * 1024,
        ),
    )(eps_key, xt, fc1_w, fc1_b, fc2_w, fc2_b, fc31_w, fc31_b, fc32_w,
      fc32_b, fc4_w, fc4_b, fc5_w, fc5_b, fc6_w, fc6_b)

    return recon, mu, lv, z


# speedup vs baseline: 1.0598x; 1.0598x over previous
import jax
import jax.numpy as jnp
import numpy as np
from jax.experimental import pallas as pl
from jax.experimental.pallas import tpu as pltpu

_PIXELS = 4096
_HIDDEN_P = 1280
_LATENT_P = 128
_LATENT = 4

_U32 = jnp.uint32

_ERFINV_CENTRAL = [2.81022636e-08, 3.43273939e-07, -3.5233877e-06,
                   -4.39150654e-06, 0.00021858087, -0.00125372503,
                   -0.00417768164, 0.246640727, 1.50140941]
_ERFINV_TAIL = [-0.000200214257, 0.000100950558, 0.00134934322,
                -0.00367342844, 0.00573950773, -0.0076224613,
                0.00943887047, 1.00167406, 2.83297682]


def _rotl(x, d):
    return (x << _U32(d)) | (x >> _U32(32 - d))


def _threefry2x32(k1, k2, x0, x1):
    ks2 = k1 ^ k2 ^ _U32(0x1BD11BDA)
    x0 = x0 + k1
    x1 = x1 + k2
    r_even = (13, 15, 26, 6)
    r_odd = (17, 29, 16, 24)
    sched = ((r_even, k2, ks2, 1), (r_odd, ks2, k1, 2), (r_even, k1, k2, 3),
             (r_odd, k2, ks2, 4), (r_even, ks2, k1, 5))
    for rots, a0, a1, inc in sched:
        for r in rots:
            x0 = x0 + x1
            x1 = _rotl(x1, r)
            x1 = x0 ^ x1
        x0 = x0 + a0
        x1 = x1 + a1 + _U32(inc)
    return x0, x1


def _eps_tile(key_ref, row0, bm):
    k1 = key_ref[0]
    k2 = key_ref[1]
    r = jax.lax.broadcasted_iota(_U32, (bm, _LATENT_P), 0)
    c = jax.lax.broadcasted_iota(_U32, (bm, _LATENT_P), 1)
    n = _U32(_LATENT) * (row0.astype(_U32) + r) + c
    b0, b1 = _threefry2x32(k1, k2, jnp.zeros_like(n), n)
    bits = b0 ^ b1
    fb = (bits >> _U32(9)) | _U32(0x3F800000)
    f = jax.lax.bitcast_convert_type(fb, jnp.float32) - jnp.float32(1.0)
    lo = np.nextafter(np.float32(-1), np.float32(0), dtype=np.float32)
    u = jnp.maximum(lo, f * (np.float32(1.0) - lo) + lo)
    w = -jnp.log1p(-u * u)
    wa = w - jnp.float32(2.5)
    wb = jnp.sqrt(w) - jnp.float32(3.0)
    pa = jnp.float32(_ERFINV_CENTRAL[0])
    for coef in _ERFINV_CENTRAL[1:]:
        pa = pa * wa + jnp.float32(coef)
    pb = jnp.float32(_ERFINV_TAIL[0])
    for coef in _ERFINV_TAIL[1:]:
        pb = pb * wb + jnp.float32(coef)
    p = jnp.where(w < jnp.float32(5.0), pa, pb)
    eps = np.float32(np.sqrt(2)) * (p * u)
    return jnp.where(c < _U32(_LATENT), eps, jnp.float32(0.0))


def _vae_kernel(key_ref, x_ref, w1_ref, b1_ref, w2_hbm, b2_ref, w31_ref,
                b31_ref, w32_ref, b32_ref, w4_ref, b4_ref, w5_hbm, b5_ref,
                w6_hbm, b6_ref, recon_ref, mu_ref, lv_ref, z_ref,
                w2_vmem, w5_vmem, w6_vmem, sem2, sem5, sem6):
    f32 = jnp.float32
    i = pl.program_id(0)
    bm = recon_ref.shape[0]

    cp2 = pltpu.make_async_copy(w2_hbm, w2_vmem, sem2)
    cp5 = pltpu.make_async_copy(w5_hbm, w5_vmem, sem5)
    cp6 = pltpu.make_async_copy(w6_hbm, w6_vmem, sem6)

    @pl.when(i == 0)
    def _start_weight_fetch():
        cp2.start()
        cp5.start()
        cp6.start()

    xb = x_ref[...].astype(jnp.bfloat16)
    h = jax.lax.dot_general(xb, w1_ref[...], (((0,), (0,)), ((), ())),
                            preferred_element_type=f32) + b1_ref[...]
    h = jnp.maximum(h, 0.0).astype(jnp.bfloat16)

    @pl.when(i == 0)
    def _wait_w2_fetch():
        cp2.wait()

    h = jnp.dot(h, w2_vmem[...], preferred_element_type=f32) + b2_ref[...]
    h = jnp.maximum(h, 0.0).astype(jnp.bfloat16)
    mu = jnp.dot(h, w31_ref[...], preferred_element_type=f32) + b31_ref[...]
    lv = jnp.dot(h, w32_ref[...], preferred_element_type=f32) + b32_ref[...]
    eps = _eps_tile(key_ref, i * bm, bm)
    z = mu + eps * jnp.exp(0.5 * lv)
    h4 = jnp.dot(z.astype(jnp.bfloat16), w4_ref[...],
                 preferred_element_type=f32) + b4_ref[...]
    h4 = jnp.maximum(h4, 0.0).astype(jnp.bfloat16)

    @pl.when(i == 0)
    def _wait_decoder_fetch():
        cp5.wait()
        cp6.wait()

    h5 = jnp.dot(h4, w5_vmem[...], preferred_element_type=f32) + b5_ref[...]
    h5 = jnp.maximum(h5, 0.0).astype(jnp.bfloat16)
    recon = jnp.dot(h5, w6_vmem[...], preferred_element_type=f32) + b6_ref[...]
    recon_ref[...] = jax.nn.sigmoid(recon)
    mu_ref[...] = mu[:, :_LATENT]
    lv_ref[...] = lv[:, :_LATENT]
    z_ref[...] = z[:, :_LATENT]


def kernel(fc1_w, fc1_b, fc2_w, fc2_b, fc31_w, fc31_b, fc32_w, fc32_b,
           fc4_w, fc4_b, fc5_w, fc5_b, fc6_w, fc6_b, x, eps_key):
    B = x.shape[0]
    xt = x.reshape(B, _PIXELS).T

    bm = 256
    while B % bm:
        bm //= 2
    grid = (B // bm,)

    row = lambda i: (i, 0)
    col = lambda i: (0, i)
    const = lambda i: (0, 0)

    recon, mu, lv, z = pl.pallas_call(
        _vae_kernel,
        out_shape=(
            jax.ShapeDtypeStruct((B, _PIXELS), jnp.float32),
            jax.ShapeDtypeStruct((B, _LATENT), jnp.float32),
            jax.ShapeDtypeStruct((B, _LATENT), jnp.float32),
            jax.ShapeDtypeStruct((B, _LATENT), jnp.float32),
        ),
        grid=grid,
        in_specs=[
            pl.BlockSpec(memory_space=pltpu.SMEM),
            pl.BlockSpec((_PIXELS, bm), col),
            pl.BlockSpec((_PIXELS, _HIDDEN_P), const),
            pl.BlockSpec((1, _HIDDEN_P), const),
            pl.BlockSpec(memory_space=pl.ANY),
            pl.BlockSpec((1, _HIDDEN_P), const),
            pl.BlockSpec((_HIDDEN_P, _LATENT_P), const),
            pl.BlockSpec((1, _LATENT_P), const),
            pl.BlockSpec((_HIDDEN_P, _LATENT_P), const),
            pl.BlockSpec((1, _LATENT_P), const),
            pl.BlockSpec((_LATENT_P, _HIDDEN_P), const),
            pl.BlockSpec((1, _HIDDEN_P), const),
            pl.BlockSpec(memory_space=pl.ANY),
            pl.BlockSpec((1, _HIDDEN_P), const),
            pl.BlockSpec(memory_space=pl.ANY),
            pl.BlockSpec((1, _PIXELS), const),
        ],
        out_specs=(
            pl.BlockSpec((bm, _PIXELS), row),
            pl.BlockSpec((bm, _LATENT), row),
            pl.BlockSpec((bm, _LATENT), row),
            pl.BlockSpec((bm, _LATENT), row),
        ),
        scratch_shapes=[
            pltpu.VMEM((_HIDDEN_P, _HIDDEN_P), jnp.bfloat16),
            pltpu.VMEM((_HIDDEN_P, _HIDDEN_P), jnp.bfloat16),
            pltpu.VMEM((_HIDDEN_P, _PIXELS), jnp.bfloat16),
            pltpu.SemaphoreType.DMA,
            pltpu.SemaphoreType.DMA,
            pltpu.SemaphoreType.DMA,
        ],
        compiler_params=pltpu.CompilerParams(
            dimension_semantics=("arbitrary",),
            vmem_limit_bytes=56 * 1024 * 1024,
        ),
    )(eps_key, xt, fc1_w, fc1_b, fc2_w, fc2_b, fc31_w, fc31_b, fc32_w,
      fc32_b, fc4_w, fc4_b, fc5_w, fc5_b, fc6_w, fc6_b)

    return recon, mu, lv, z
